# baseline (speedup 1.0000x reference)
import functools

import jax
import jax.numpy as jnp
from jax import lax
from jax.experimental import pallas as pl
from jax.experimental.pallas import tpu as pltpu
from jax.experimental.pallas import tpu_sc as plsc

_B = 16384
_D = 32
_NC = 2
_NS = 16
_NW = _NC * _NS
_BPW = _B // _NW


def _gather_kernel(uidx_hbm, iidx_hbm, utab_hbm, itab_hbm, ue_hbm, ie_hbm,
                   uix_v, iix_v, rows_v, sem):
    wid = lax.axis_index("s") * _NC + lax.axis_index("c")
    base = wid * _BPW
    pltpu.sync_copy(uidx_hbm.at[wid], uix_v)
    pltpu.sync_copy(iidx_hbm.at[wid], iix_v)

    def stage(idx_v, tab_hbm, out_hbm):
        def grp(g):
            vec = idx_v[pl.ds(g * 16, 16)] >> 2
            for k in range(16):
                r = vec[k]
                pltpu.async_copy(tab_hbm.at[pl.ds(r, 1)],
                                 rows_v.at[pl.ds(g * 16 + k, 1)], sem)
        pl.loop(0, _BPW // 16)(grp)
        pltpu.make_async_copy(tab_hbm.at[pl.ds(0, _BPW)], rows_v, sem).wait()
        pltpu.sync_copy(rows_v, out_hbm.at[pl.ds(base, _BPW)])

    stage(uix_v, utab_hbm, ue_hbm)
    stage(iix_v, itab_hbm, ie_hbm)


@jax.jit
def _gather(uidx, iidx, user_table4, item_table4):
    mesh = plsc.VectorSubcoreMesh(core_axis_name="c", subcore_axis_name="s")
    return pl.kernel(
        _gather_kernel,
        mesh=mesh,
        compiler_params=pltpu.CompilerParams(use_tc_tiling_on_sc=True),
        out_type=(
            jax.ShapeDtypeStruct((_B, 128), jnp.float32),
            jax.ShapeDtypeStruct((_B, 128), jnp.float32),
        ),
        scratch_types=[
            pltpu.VMEM((_BPW,), jnp.int32),
            pltpu.VMEM((_BPW,), jnp.int32),
            pltpu.VMEM((_BPW, 128), jnp.float32),
            pltpu.SemaphoreType.DMA,
        ],
    )(uidx, iidx, user_table4, item_table4)


_BLK = 2048


def _mlp_kernel(ue4_ref, ie4_ref, u_ref, i_ref, w1a_ref, w1b_ref, b1_ref,
                w2_ref, b2_ref, w3_ref, b3_ref, out_ref):
    def pick(rows4, idx):
        p = idx & 3
        x = rows4[:, 0:_D]
        for k in range(1, 4):
            x = jnp.where(p == k, rows4[:, k * _D:(k + 1) * _D], x)
        return x

    ue = pick(ue4_ref[...], u_ref[...])
    ie = pick(ie4_ref[...], i_ref[...])
    x = (jnp.dot(ue, w1a_ref[...], preferred_element_type=jnp.float32)
         + jnp.dot(ie, w1b_ref[...], preferred_element_type=jnp.float32)
         + b1_ref[...])
    h1 = jnp.maximum(x, 0.0)
    h2 = jnp.maximum(
        jnp.dot(h1, w2_ref[...], preferred_element_type=jnp.float32)
        + b2_ref[...], 0.0)
    logit = jnp.sum(h2 * w3_ref[...], axis=1) + b3_ref[0, 0]
    out_ref[...] = jax.nn.sigmoid(logit)


@jax.jit
def _mlp(ue4, ie4, users, items, w1a, w1b, b1, w2, b2, w3, b3):
    grid = (_B // _BLK,)
    full = lambda i: (0, 0)
    return pl.pallas_call(
        _mlp_kernel,
        grid=grid,
        in_specs=[
            pl.BlockSpec((_BLK, 128), lambda i: (i, 0)),
            pl.BlockSpec((_BLK, 128), lambda i: (i, 0)),
            pl.BlockSpec((_BLK, 1), lambda i: (i, 0)),
            pl.BlockSpec((_BLK, 1), lambda i: (i, 0)),
            pl.BlockSpec((_D, 128), full),
            pl.BlockSpec((_D, 128), full),
            pl.BlockSpec((1, 128), full),
            pl.BlockSpec((128, 64), full),
            pl.BlockSpec((1, 64), full),
            pl.BlockSpec((1, 64), full),
            pl.BlockSpec((1, 1), full),
        ],
        out_specs=pl.BlockSpec((_BLK,), lambda i: (i,)),
        out_shape=jax.ShapeDtypeStruct((_B,), jnp.float32),
    )(ue4, ie4, users, items, w1a, w1b, b1, w2, b2, w3, b3)


def kernel(users, items, user_table, item_table, W1, b1, W2, b2, W3, b3):
    uidx = users.reshape(_NW, _BPW)
    iidx = items.reshape(_NW, _BPW)
    ut4 = user_table.reshape(250000, 128)
    it4 = item_table.reshape(250000, 128)
    ue4, ie4 = _gather(uidx, iidx, ut4, it4)
    return _mlp(ue4, ie4, users.reshape(_B, 1), items.reshape(_B, 1),
                W1[:_D], W1[_D:], b1.reshape(1, 128),
                W2, b2.reshape(1, 64), W3.reshape(1, 64), b3.reshape(1, 1))
